# baseline (speedup 1.0000x reference)
import functools

import jax
import jax.numpy as jnp
from jax import lax
from jax.experimental import pallas as pl
from jax.experimental.pallas import tpu as pltpu
from jax.experimental.pallas import tpu_sc as plsc

_NC = 2
_NS = 16
_NW = _NC * _NS
_LANES = 16
_CH = 8


def _coef_body(u_ref, a_ref, w_ref, c_ref):
    u = u_ref[...]
    t = jnp.dot(a_ref[...], u, precision=lax.Precision.HIGHEST,
                preferred_element_type=jnp.float32)
    s = t * w_ref[...]
    c_ref[...] = lax.dot_general(
        s, u, (((1,), (1,)), ((), ())), precision=lax.Precision.HIGHEST,
        preferred_element_type=jnp.float32)


@functools.partial(jax.jit, static_argnums=(0,))
def _coefficients(K, weight, avgweight, U):
    u_pad = jnp.zeros((128, 128), jnp.float32).at[:K, :K].set(U)
    a_row = jnp.zeros((8, 128), jnp.float32).at[0, :K].set(avgweight[:, 0])
    w_row = jnp.zeros((8, 128), jnp.float32).at[0, :K].set(weight[:, 0])
    return pl.pallas_call(
        _coef_body,
        out_shape=jax.ShapeDtypeStruct((8, 128), jnp.float32),
    )(u_pad, a_row, w_row)


def _make_sc_kernel(b_pad, d, s_slots, n_chunks, nbuf):
    mesh = plsc.VectorSubcoreMesh(core_axis_name="c", subcore_axis_name="s")
    grp = _CH * s_slots

    scratch = [pltpu.VMEM((n_chunks, grp), jnp.int32)]
    scratch += [pltpu.VMEM((grp, d // 2), jnp.int32) for _ in range(nbuf)]
    scratch += [pltpu.VMEM((_CH, d), jnp.float32) for _ in range(nbuf)]
    scratch += [pltpu.VMEM((s_slots, _LANES), jnp.float32)]
    scratch += [pltpu.SemaphoreType.DMA for _ in range(2 * nbuf)]

    @functools.partial(
        pl.kernel,
        mesh=mesh,
        out_type=jax.ShapeDtypeStruct((b_pad, d), jnp.float32),
        scratch_types=scratch,
        compiler_params=pltpu.CompilerParams(use_tc_tiling_on_sc=False),
    )
    def sc_k(idx_hbm, table_hbm, cb_hbm, out_hbm, *sc):
        idx_v = sc[0]
        rows = sc[1:1 + nbuf]
        outs = sc[1 + nbuf:1 + 2 * nbuf]
        cb_v = sc[1 + 2 * nbuf]
        sgs = sc[2 + 2 * nbuf:2 + 3 * nbuf]
        sos = sc[2 + 3 * nbuf:2 + 4 * nbuf]
        wid = lax.axis_index("s") * _NC + lax.axis_index("c")
        pltpu.sync_copy(cb_hbm, cb_v)
        pltpu.sync_copy(idx_hbm.at[wid], idx_v)
        nv = d // _LANES

        def compute(rv, ov):
            zero = jnp.zeros((_LANES,), jnp.float32)
            nw = d // 32
            for r0 in range(0, _CH, 2):
                def s_body(s, accs):
                    cs = cb_v[s, :]
                    accs = list(accs)
                    hi_mask = jnp.full((_LANES,), -65536, jnp.int32)
                    for rr in range(2):
                        for k in range(nw):
                            w = rv[(r0 + rr) * s_slots + s,
                                   pl.ds(k * _LANES, _LANES)]
                            a = lax.bitcast_convert_type(
                                w << 16, jnp.float32)
                            b = lax.bitcast_convert_type(
                                w & hi_mask, jnp.float32)
                            i = rr * 2 * nw + 2 * k
                            accs[i] = accs[i] + cs * a
                            accs[i + 1] = accs[i + 1] + cs * b
                    return tuple(accs)
                accs = lax.fori_loop(0, s_slots, s_body,
                                     (zero,) * (4 * nw))
                for rr in range(2):
                    for k in range(nw):
                        i = rr * 2 * nw + 2 * k
                        ov[r0 + rr, pl.ds(32 * k, _LANES)] = accs[i]
                        ov[r0 + rr, pl.ds(32 * k + _LANES, _LANES)] = \
                            accs[i + 1]

        for b in range(nbuf):
            pltpu.async_copy(table_hbm.at[idx_v.at[b]], rows[b], sgs[b])

        def grp_body(jg, carry):
            for b in range(nbuf):
                j = nbuf * jg + b
                pltpu.make_async_copy(
                    table_hbm.at[idx_v.at[j]], rows[b], sgs[b]).wait()

                @pl.when(jg > 0)
                def _drain_prev_write():
                    pltpu.make_async_copy(
                        outs[b], out_hbm.at[pl.ds(0, _CH)], sos[b]).wait()

                compute(rows[b], outs[b])
                pltpu.async_copy(
                    outs[b],
                    out_hbm.at[pl.ds((wid + _NW * j) * _CH, _CH)], sos[b])

                @pl.when(j + nbuf < n_chunks)
                def _start_next_gather():
                    pltpu.async_copy(
                        table_hbm.at[idx_v.at[j + nbuf]], rows[b], sgs[b])

            return carry

        lax.fori_loop(0, n_chunks // nbuf, grp_body, 0)
        for b in range(nbuf):
            pltpu.make_async_copy(
                outs[b], out_hbm.at[pl.ds(0, _CH)], sos[b]).wait()

    return sc_k


def kernel(feat_table, neighbor_idx, weight, avgweight, U):
    B, S = neighbor_idx.shape
    D = feat_table.shape[1]
    K = U.shape[0]

    c_row = _coefficients(K, weight, avgweight, U)
    cb = jnp.broadcast_to(c_row[0, 1:1 + S].reshape(S, 1), (S, _LANES))

    nbuf = 2
    step = _NW * _CH * nbuf
    b_pad = ((B + step - 1) // step) * step
    n_chunks = b_pad // (_NW * _CH)
    idx = neighbor_idx.astype(jnp.int32)
    idx_p = jnp.zeros((b_pad, S), jnp.int32).at[:B].set(idx)
    idx_r = jnp.transpose(
        idx_p.reshape(n_chunks, _NW, _CH * S), (1, 0, 2))

    col = jnp.arange(D)
    blk, r = col // 32, col % 32
    src = 32 * blk + jnp.where(r % 2 == 0, r // 2, 16 + r // 2)
    t_pack = jax.lax.bitcast_convert_type(
        feat_table[:, src].astype(jnp.bfloat16).reshape(-1, D // 2, 2),
        jnp.int32)

    out_p = _make_sc_kernel(b_pad, D, S, n_chunks, nbuf)(idx_r, t_pack, cb)
    return out_p[:B]


# speedup vs baseline: 2.3585x; 2.3585x over previous
import functools

import jax
import jax.numpy as jnp
from jax import lax
from jax.experimental import pallas as pl
from jax.experimental.pallas import tpu as pltpu
from jax.experimental.pallas import tpu_sc as plsc

_NC = 2
_NS = 16
_NW = _NC * _NS
_LANES = 16
_CH = 8


def _coef_body(u_ref, a_ref, w_ref, c_ref):
    u = u_ref[...]
    t = jnp.dot(a_ref[...], u, precision=lax.Precision.HIGHEST,
                preferred_element_type=jnp.float32)
    s = t * w_ref[...]
    c_ref[...] = lax.dot_general(
        s, u, (((1,), (1,)), ((), ())), precision=lax.Precision.HIGHEST,
        preferred_element_type=jnp.float32)


@functools.partial(jax.jit, static_argnums=(0,))
def _coefficients(K, weight, avgweight, U):
    u_pad = jnp.zeros((128, 128), jnp.float32).at[:K, :K].set(U)
    a_row = jnp.zeros((8, 128), jnp.float32).at[0, :K].set(avgweight[:, 0])
    w_row = jnp.zeros((8, 128), jnp.float32).at[0, :K].set(weight[:, 0])
    return pl.pallas_call(
        _coef_body,
        out_shape=jax.ShapeDtypeStruct((8, 128), jnp.float32),
    )(u_pad, a_row, w_row)


def _make_sc_kernel(b_pad, d, s_slots, n_chunks, nbuf):
    mesh = plsc.VectorSubcoreMesh(core_axis_name="c", subcore_axis_name="s")
    grp = _CH * s_slots

    scratch = [pltpu.VMEM((n_chunks, grp), jnp.int32)]
    scratch += [pltpu.VMEM((grp, d), jnp.float32) for _ in range(nbuf)]
    scratch += [pltpu.VMEM((_CH, d), jnp.float32) for _ in range(nbuf)]
    scratch += [pltpu.VMEM((s_slots, _LANES), jnp.float32)]
    scratch += [pltpu.SemaphoreType.DMA for _ in range(2 * nbuf)]

    @functools.partial(
        pl.kernel,
        mesh=mesh,
        out_type=jax.ShapeDtypeStruct((b_pad, d), jnp.float32),
        scratch_types=scratch,
    )
    def sc_k(idx_hbm, table_hbm, cb_hbm, out_hbm, *sc):
        idx_v = sc[0]
        rows = sc[1:1 + nbuf]
        outs = sc[1 + nbuf:1 + 2 * nbuf]
        cb_v = sc[1 + 2 * nbuf]
        sgs = sc[2 + 2 * nbuf:2 + 3 * nbuf]
        sos = sc[2 + 3 * nbuf:2 + 4 * nbuf]
        wid = lax.axis_index("s") * _NC + lax.axis_index("c")
        pltpu.sync_copy(cb_hbm, cb_v)
        pltpu.sync_copy(idx_hbm.at[:, wid], idx_v)
        nv = d // _LANES

        def compute(rv, ov):
            zero = jnp.zeros((_LANES,), jnp.float32)
            for r0 in range(0, _CH, 2):
                def s_body(s, accs):
                    cs = cb_v[s, :]
                    return tuple(
                        accs[i] + cs * rv[(r0 + i // nv) * s_slots + s,
                                          pl.ds((i % nv) * _LANES, _LANES)]
                        for i in range(2 * nv))
                accs = lax.fori_loop(0, s_slots, s_body, (zero,) * (2 * nv))
                for i in range(2 * nv):
                    ov[r0 + i // nv, pl.ds((i % nv) * _LANES, _LANES)] = \
                        accs[i]

        for b in range(nbuf):
            pltpu.async_copy(table_hbm.at[idx_v.at[b]], rows[b], sgs[b])

        def grp_body(jg, carry):
            for b in range(nbuf):
                j = nbuf * jg + b
                pltpu.make_async_copy(
                    table_hbm.at[idx_v.at[j]], rows[b], sgs[b]).wait()

                @pl.when(jg > 0)
                def _drain_prev_write():
                    pltpu.make_async_copy(
                        outs[b], out_hbm.at[pl.ds(0, _CH)], sos[b]).wait()

                compute(rows[b], outs[b])
                pltpu.async_copy(
                    outs[b],
                    out_hbm.at[pl.ds((wid + _NW * j) * _CH, _CH)], sos[b])

                @pl.when(j + nbuf < n_chunks)
                def _start_next_gather():
                    pltpu.async_copy(
                        table_hbm.at[idx_v.at[j + nbuf]], rows[b], sgs[b])

            return carry

        lax.fori_loop(0, n_chunks // nbuf, grp_body, 0)
        for b in range(nbuf):
            pltpu.make_async_copy(
                outs[b], out_hbm.at[pl.ds(0, _CH)], sos[b]).wait()

    return sc_k


def kernel(feat_table, neighbor_idx, weight, avgweight, U):
    B, S = neighbor_idx.shape
    D = feat_table.shape[1]
    K = U.shape[0]

    c_row = _coefficients(K, weight, avgweight, U)
    cb = jnp.broadcast_to(c_row[0, 1:1 + S].reshape(S, 1), (S, _LANES))

    nbuf = 2
    step = _NW * _CH * nbuf
    b_pad = ((B + step - 1) // step) * step
    n_chunks = b_pad // (_NW * _CH)
    idx = neighbor_idx.astype(jnp.int32)
    idx_p = jnp.zeros((b_pad, S), jnp.int32).at[:B].set(idx)
    idx_r = idx_p.reshape(n_chunks, _NW, _CH * S)

    out_p = _make_sc_kernel(b_pad, D, S, n_chunks, nbuf)(idx_r, feat_table,
                                                         cb)
    return out_p[:B]


# speedup vs baseline: 2.5180x; 1.0676x over previous
import functools

import jax
import jax.numpy as jnp
from jax import lax
from jax.experimental import pallas as pl
from jax.experimental.pallas import tpu as pltpu
from jax.experimental.pallas import tpu_sc as plsc

_NC = 2
_NS = 16
_NW = _NC * _NS
_LANES = 16
_CH = 8


def _coef_body(u_ref, a_ref, w_ref, c_ref):
    u = u_ref[...]
    t = jnp.dot(a_ref[...], u, precision=lax.Precision.HIGHEST,
                preferred_element_type=jnp.float32)
    s = t * w_ref[...]
    c_ref[...] = lax.dot_general(
        s, u, (((1,), (1,)), ((), ())), precision=lax.Precision.HIGHEST,
        preferred_element_type=jnp.float32)


@functools.partial(jax.jit, static_argnums=(0,))
def _coefficients(K, weight, avgweight, U):
    u_pad = jnp.zeros((128, 128), jnp.float32).at[:K, :K].set(U)
    a_row = jnp.zeros((8, 128), jnp.float32).at[0, :K].set(avgweight[:, 0])
    w_row = jnp.zeros((8, 128), jnp.float32).at[0, :K].set(weight[:, 0])
    return pl.pallas_call(
        _coef_body,
        out_shape=jax.ShapeDtypeStruct((8, 128), jnp.float32),
    )(u_pad, a_row, w_row)


def _make_sc_kernel(b_real, d, s_slots, n_chunks, nbuf):
    mesh = plsc.VectorSubcoreMesh(core_axis_name="c", subcore_axis_name="s")
    grp = _CH * s_slots

    scratch = [pltpu.VMEM((n_chunks, grp), jnp.int32)]
    scratch += [pltpu.VMEM((grp, d), jnp.float32) for _ in range(nbuf)]
    scratch += [pltpu.VMEM((_CH, d), jnp.float32) for _ in range(nbuf)]
    scratch += [pltpu.VMEM((s_slots, _LANES), jnp.float32)]
    scratch += [pltpu.SemaphoreType.DMA for _ in range(2 * nbuf)]

    @functools.partial(
        pl.kernel,
        mesh=mesh,
        out_type=jax.ShapeDtypeStruct((b_real, d), jnp.float32),
        scratch_types=scratch,
    )
    def sc_k(idx_hbm, table_hbm, cb_hbm, out_hbm, *sc):
        idx_v = sc[0]
        rows = sc[1:1 + nbuf]
        outs = sc[1 + nbuf:1 + 2 * nbuf]
        cb_v = sc[1 + 2 * nbuf]
        sgs = sc[2 + 2 * nbuf:2 + 3 * nbuf]
        sos = sc[2 + 3 * nbuf:2 + 4 * nbuf]
        wid = lax.axis_index("s") * _NC + lax.axis_index("c")
        pltpu.sync_copy(cb_hbm, cb_v)
        pltpu.sync_copy(idx_hbm.at[:, wid], idx_v)
        nv = d // _LANES

        def compute(rv, ov):
            zero = jnp.zeros((_LANES,), jnp.float32)
            for r0 in range(0, _CH, 2):
                def s_body(s, accs):
                    cs = cb_v[s, :]
                    return tuple(
                        accs[i] + cs * rv[(r0 + i // nv) * s_slots + s,
                                          pl.ds((i % nv) * _LANES, _LANES)]
                        for i in range(2 * nv))
                accs = lax.fori_loop(0, s_slots, s_body, (zero,) * (2 * nv))
                for i in range(2 * nv):
                    ov[r0 + i // nv, pl.ds((i % nv) * _LANES, _LANES)] = \
                        accs[i]

        for b in range(nbuf):
            pltpu.async_copy(table_hbm.at[idx_v.at[b]], rows[b], sgs[b])

        def grp_body(jg, carry):
            for b in range(nbuf):
                j = nbuf * jg + b
                pltpu.make_async_copy(
                    table_hbm.at[idx_v.at[j]], rows[b], sgs[b]).wait()

                @pl.when(jg > 0)
                def _drain_prev_write():
                    pltpu.make_async_copy(
                        outs[b], out_hbm.at[pl.ds(0, _CH)], sos[b]).wait()

                compute(rows[b], outs[b])
                base = (wid + _NW * j) * _CH

                @pl.when(base < b_real)
                def _store():
                    pltpu.async_copy(
                        outs[b], out_hbm.at[pl.ds(base, _CH)], sos[b])

                @pl.when(j + nbuf < n_chunks)
                def _start_next_gather():
                    pltpu.async_copy(
                        table_hbm.at[idx_v.at[j + nbuf]], rows[b], sgs[b])

            return carry

        lax.fori_loop(0, n_chunks // nbuf, grp_body, 0)
        for b in range(nbuf):
            @pl.when((wid + _NW * (n_chunks - nbuf + b)) * _CH < b_real)
            def _final_drain():
                pltpu.make_async_copy(
                    outs[b], out_hbm.at[pl.ds(0, _CH)], sos[b]).wait()

    return sc_k


def kernel(feat_table, neighbor_idx, weight, avgweight, U):
    B, S = neighbor_idx.shape
    D = feat_table.shape[1]
    K = U.shape[0]

    c_row = _coefficients(K, weight, avgweight, U)
    cb = jnp.broadcast_to(c_row[0, 1:1 + S].reshape(S, 1), (S, _LANES))

    nbuf = 2
    step = _NW * _CH * nbuf
    b_pad = ((B + step - 1) // step) * step
    n_chunks = b_pad // (_NW * _CH)
    grp = _CH * S
    idx_flat = jnp.zeros((b_pad * S // grp, grp), jnp.int32)
    idx_flat = idx_flat.at[:B * S // grp].set(
        neighbor_idx.astype(jnp.int32).reshape(B * S // grp, grp))
    idx_r = idx_flat.reshape(n_chunks, _NW, grp)

    return _make_sc_kernel(B, D, S, n_chunks, nbuf)(idx_r, feat_table, cb)


# speedup vs baseline: 2.5240x; 1.0024x over previous
import functools

import jax
import jax.numpy as jnp
from jax import lax
from jax.experimental import pallas as pl
from jax.experimental.pallas import tpu as pltpu
from jax.experimental.pallas import tpu_sc as plsc

_NC = 2
_NS = 16
_NW = _NC * _NS
_LANES = 16
_CH = 8


def _coef_body(u_ref, a_ref, w_ref, c_ref):
    u = u_ref[...]
    t = jnp.dot(a_ref[...], u, precision=lax.Precision.HIGHEST,
                preferred_element_type=jnp.float32)
    s = t * w_ref[...]
    c_ref[...] = lax.dot_general(
        s, u, (((1,), (1,)), ((), ())), precision=lax.Precision.HIGHEST,
        preferred_element_type=jnp.float32)


@functools.partial(jax.jit, static_argnums=(0,))
def _coefficients(K, weight, avgweight, U):
    u_pad = jnp.zeros((128, 128), jnp.float32).at[:K, :K].set(U)
    a_row = jnp.zeros((8, 128), jnp.float32).at[0, :K].set(avgweight[:, 0])
    w_row = jnp.zeros((8, 128), jnp.float32).at[0, :K].set(weight[:, 0])
    return pl.pallas_call(
        _coef_body,
        out_shape=jax.ShapeDtypeStruct((8, 128), jnp.float32),
    )(u_pad, a_row, w_row)


def _make_sc_kernel(b_real, d, s_slots, n_chunks, nbuf):
    mesh = plsc.VectorSubcoreMesh(core_axis_name="c", subcore_axis_name="s")
    grp = _CH * s_slots

    scratch = [pltpu.VMEM((n_chunks, grp), jnp.int32)]
    scratch += [pltpu.VMEM((grp, d), jnp.float32) for _ in range(nbuf)]
    scratch += [pltpu.VMEM((_CH, d), jnp.float32) for _ in range(nbuf)]
    scratch += [pltpu.VMEM((s_slots, _LANES), jnp.float32)]
    scratch += [pltpu.SemaphoreType.DMA for _ in range(2 * nbuf)]

    @functools.partial(
        pl.kernel,
        mesh=mesh,
        out_type=jax.ShapeDtypeStruct((b_real, d), jnp.float32),
        scratch_types=scratch,
    )
    def sc_k(idx_hbm, table_hbm, cb_hbm, out_hbm, *sc):
        idx_v = sc[0]
        rows = sc[1:1 + nbuf]
        outs = sc[1 + nbuf:1 + 2 * nbuf]
        cb_v = sc[1 + 2 * nbuf]
        sgs = sc[2 + 2 * nbuf:2 + 3 * nbuf]
        sos = sc[2 + 3 * nbuf:2 + 4 * nbuf]
        wid = lax.axis_index("c") * _NS + lax.axis_index("s")
        pltpu.sync_copy(cb_hbm, cb_v)
        pltpu.sync_copy(idx_hbm.at[:, wid], idx_v)
        nv = d // _LANES

        def compute(rv, ov):
            zero = jnp.zeros((_LANES,), jnp.float32)
            for r0 in range(0, _CH, 2):
                def s_body(s, accs):
                    cs = cb_v[s, :]
                    return tuple(
                        accs[i] + cs * rv[(r0 + i // nv) * s_slots + s,
                                          pl.ds((i % nv) * _LANES, _LANES)]
                        for i in range(2 * nv))
                accs = lax.fori_loop(0, s_slots, s_body, (zero,) * (2 * nv))
                for i in range(2 * nv):
                    ov[r0 + i // nv, pl.ds((i % nv) * _LANES, _LANES)] = \
                        accs[i]

        for b in range(nbuf):
            pltpu.async_copy(table_hbm.at[idx_v.at[b]], rows[b], sgs[b])

        def grp_body(jg, carry):
            for b in range(nbuf):
                j = nbuf * jg + b
                pltpu.make_async_copy(
                    table_hbm.at[idx_v.at[j]], rows[b], sgs[b]).wait()

                @pl.when(jg > 0)
                def _drain_prev_write():
                    pltpu.make_async_copy(
                        outs[b], out_hbm.at[pl.ds(0, _CH)], sos[b]).wait()

                compute(rows[b], outs[b])
                base = (wid + _NW * j) * _CH

                @pl.when(base < b_real)
                def _store():
                    pltpu.async_copy(
                        outs[b], out_hbm.at[pl.ds(base, _CH)], sos[b])

                @pl.when(j + nbuf < n_chunks)
                def _start_next_gather():
                    pltpu.async_copy(
                        table_hbm.at[idx_v.at[j + nbuf]], rows[b], sgs[b])

            return carry

        lax.fori_loop(0, n_chunks // nbuf, grp_body, 0)
        for b in range(nbuf):
            @pl.when((wid + _NW * (n_chunks - nbuf + b)) * _CH < b_real)
            def _final_drain():
                pltpu.make_async_copy(
                    outs[b], out_hbm.at[pl.ds(0, _CH)], sos[b]).wait()

    return sc_k


def kernel(feat_table, neighbor_idx, weight, avgweight, U):
    B, S = neighbor_idx.shape
    D = feat_table.shape[1]
    K = U.shape[0]

    c_row = _coefficients(K, weight, avgweight, U)
    cb = jnp.broadcast_to(c_row[0, 1:1 + S].reshape(S, 1), (S, _LANES))

    nbuf = 2
    step = _NW * _CH * nbuf
    b_pad = ((B + step - 1) // step) * step
    n_chunks = b_pad // (_NW * _CH)
    grp = _CH * S
    idx_flat = jnp.zeros((b_pad * S // grp, grp), jnp.int32)
    idx_flat = idx_flat.at[:B * S // grp].set(
        neighbor_idx.astype(jnp.int32).reshape(B * S // grp, grp))
    idx_r = idx_flat.reshape(n_chunks, _NW, grp)

    return _make_sc_kernel(B, D, S, n_chunks, nbuf)(idx_r, feat_table, cb)
